# baseline (speedup 1.0000x reference)
import jax
import jax.numpy as jnp
from jax.experimental import pallas as pl
from jax.experimental.pallas import tpu as pltpu

_BM = 400


def _pass1_body(adj_ref, xb_ref, w1_ref, b1_ref, h_ref, adjb_ref):
    ab = adj_ref[...].astype(jnp.bfloat16)
    adjb_ref[...] = ab
    pre = jnp.dot(ab, xb_ref[...], preferred_element_type=jnp.float32)
    hblk = jax.lax.dot_general(
        pre.astype(jnp.bfloat16), w1_ref[...].astype(jnp.bfloat16),
        (((1,), (1,)), ((), ())),
        preferred_element_type=jnp.float32) + b1_ref[...]
    h_ref[...] = jnp.maximum(hblk, 0.0).astype(jnp.bfloat16)


def _pass2_body(adjb_ref, h_ref, w2_ref, b2_ref, y_ref, emb_ref):
    emb = jnp.dot(adjb_ref[...], h_ref[...], preferred_element_type=jnp.float32)
    emb_ref[...] = emb
    y_ref[...] = jax.lax.dot_general(
        emb.astype(jnp.bfloat16), w2_ref[...].astype(jnp.bfloat16),
        (((1,), (1,)), ((), ())),
        preferred_element_type=jnp.float32) + b2_ref[...]


def kernel(x, adj, W1, b1, W2, b2):
    n, xd = x.shape
    hd = W1.shape[0]
    yd = W2.shape[0]
    nb = n // _BM

    h, adjb = pl.pallas_call(
        _pass1_body,
        grid=(nb,),
        in_specs=[
            pl.BlockSpec((_BM, n), lambda i: (i, 0)),
            pl.BlockSpec((n, xd), lambda i: (0, 0)),
            pl.BlockSpec((hd, xd), lambda i: (0, 0)),
            pl.BlockSpec((1, hd), lambda i: (0, 0)),
        ],
        out_specs=[
            pl.BlockSpec((_BM, hd), lambda i: (i, 0)),
            pl.BlockSpec((_BM, n), lambda i: (i, 0)),
        ],
        out_shape=[
            jax.ShapeDtypeStruct((n, hd), jnp.bfloat16),
            jax.ShapeDtypeStruct((n, n), jnp.bfloat16),
        ],
        compiler_params=pltpu.CompilerParams(
            dimension_semantics=("arbitrary",),
        ),
        interpret=False,
    )(adj, x.astype(jnp.bfloat16), W1, b1.reshape(1, hd))

    y, emb = pl.pallas_call(
        _pass2_body,
        grid=(nb,),
        in_specs=[
            pl.BlockSpec((_BM, n), lambda i: (i, 0)),
            pl.BlockSpec((n, hd), lambda i: (0, 0)),
            pl.BlockSpec((yd, hd), lambda i: (0, 0)),
            pl.BlockSpec((1, yd), lambda i: (0, 0)),
        ],
        out_specs=[
            pl.BlockSpec((_BM, yd), lambda i: (i, 0)),
            pl.BlockSpec((_BM, hd), lambda i: (i, 0)),
        ],
        out_shape=[
            jax.ShapeDtypeStruct((n, yd), jnp.float32),
            jax.ShapeDtypeStruct((n, hd), jnp.float32),
        ],
        compiler_params=pltpu.CompilerParams(
            dimension_semantics=("arbitrary",),
        ),
        interpret=False,
    )(adjb, h, W2, b2.reshape(1, yd))
    return (y, emb)
